# baseline (speedup 1.0000x reference)
import functools

import jax
import jax.numpy as jnp
from jax import lax
from jax.experimental import pallas as pl
from jax.experimental.pallas import tpu as pltpu
from jax.experimental.pallas import tpu_sc as plsc

GRID_ROWS, GRID_COLS = 128, 128
SDF_SIZE = GRID_ROWS * GRID_COLS
COLS = 3 * SDF_SIZE + 6
PARAM_BASE = 3 * SDF_SIZE
BATCH = 1024
NC, NS, L = 2, 16, 16
NW = NC * NS
RPW = BATCH // NW

_mesh = plsc.VectorSubcoreMesh(core_axis_name="c", subcore_axis_name="s")


@functools.partial(
    pl.kernel,
    out_type=jax.ShapeDtypeStruct((BATCH,), jnp.float32),
    mesh=_mesh,
    scratch_types=[
        pltpu.VMEM((6 * RPW,), jnp.float32),
        pltpu.VMEM((RPW, 8, 128), jnp.float32),
        pltpu.VMEM((RPW + L,), jnp.int32),
        pltpu.VMEM((RPW + L,), jnp.int32),
        pltpu.VMEM((RPW,), jnp.float32),
        pltpu.SemaphoreType.DMA,
    ],
    compiler_params=pltpu.CompilerParams(skip_device_barrier=True),
)
def _sdf_lookup(xT, pflat, out, pvals, vbuf, cbuf, sbuf, obuf, sem):
    wid = lax.axis_index("s") * NC + lax.axis_index("c")
    base = wid * RPW
    rband = pl.multiple_of(base & ~127, 128)
    boff = base & 127
    lanes = lax.iota(jnp.int32, L)

    pltpu.async_copy(
        pflat.at[pl.ds(wid * (6 * RPW), 6 * RPW)], pvals, sem
    ).wait()

    oobs = []
    for h in range(RPW // L):
        sl = pl.ds(h * L, L)

        def p(j, _h=h):
            return pvals[pl.ds(j * RPW + _h * L, L)]

        i0 = (p(4) / p(0) + p(2)).astype(jnp.int32)
        i1 = (p(5) / p(1) + p(3)).astype(jnp.int32)
        flat = i0 * GRID_COLS + i1
        oob = (i0 < 0) | (i0 >= GRID_COLS) | (i1 < 0) | (i1 >= GRID_ROWS)
        safe = jnp.clip(flat, 0, SDF_SIZE - 1)
        cbuf[sl] = safe & ~7
        sbuf[sl] = safe & 7
        oobs.append(oob)

    def fire(i, _):
        ct = pl.multiple_of(cbuf[pl.ds(i, L)][0], 8)
        pltpu.async_copy(
            xT.at[pl.ds(ct, 8), pl.ds(rband, 128)], vbuf.at[i], sem
        )
        return _

    lax.fori_loop(0, RPW, fire, 0, unroll=False)

    def drain(i, _):
        pltpu.make_async_copy(
            xT.at[pl.ds(0, 8), pl.ds(rband, 128)], vbuf.at[i], sem
        ).wait()
        return _

    lax.fori_loop(0, RPW, drain, 0, unroll=False)

    for h in range(RPW // L):
        sl = pl.ds(h * L, L)
        off = pl.multiple_of(boff + h * L, L)

        def extract(l, acc, _h=h, _off=off):
            i = _h * L + l
            v16 = vbuf[i, sbuf[pl.ds(i, L)][0], pl.ds(_off, L)]
            return jnp.where(lanes == l, v16, acc)

        vals = lax.fori_loop(0, L, extract, jnp.full((L,), 0.0, jnp.float32),
                             unroll=False)
        obuf[sl] = jnp.where(oobs[h], jnp.float32(-0.1), vals)

    pltpu.sync_copy(obuf, out.at[pl.ds(base, RPW)])


def kernel(x):
    pflat = (
        x[:, PARAM_BASE:PARAM_BASE + 6]
        .reshape(NW, RPW, 6)
        .transpose(0, 2, 1)
        .reshape(-1)
    )
    return _sdf_lookup(x.T, pflat)[:, None]
